# baseline (speedup 1.0000x reference)
import functools

import numpy as np
import jax
import jax.numpy as jnp
from jax import lax
from jax.experimental import pallas as pl
from jax.experimental.pallas import tpu as pltpu
from jax.experimental.pallas import tpu_sc as plsc

BATCH = 16384
EMB = 128
NC, NS = 2, 16
NW = NC * NS
CHUNK = 128
PIPE = 2
CB = BATCH // PIPE
B_PER_W = CB // NW
NCH = B_PER_W // CHUNK

_PERM = np.arange(EMB).reshape(-1, 2, 16).transpose(0, 2, 1).reshape(-1)


def _convert_rows(rows_v, bf_v, j):
    def row_body(r, carry):
        for k in range(EMB // 32):
            a = rows_v[j, r, pl.ds(32 * k, 16)]
            b = rows_v[j, r, pl.ds(32 * k + 16, 16)]
            bf_v[j, r, pl.ds(32 * k, 32)] = plsc.pack(
                a, b, format=plsc.PackFormat.INTERLEAVED)
        return carry
    lax.fori_loop(0, CHUNK, row_body, 0)


def _make_gather_body(p):
    def _gather_body(users_hbm, items_hbm, utab_hbm, mtab_hbm,
                     uout_hbm, iout_hbm, idxu_v, idxi_v, rowsu_v, rowsi_v,
                     bfu_v, bfi_v, sem_u, sem_i, sem_out):
        wid = lax.axis_index("s") * NC + lax.axis_index("c")
        src = p * CB + wid * B_PER_W
        dst = wid * B_PER_W
        for j in range(NCH):
            pltpu.sync_copy(users_hbm.at[pl.ds(src + j * CHUNK, CHUNK)],
                            idxu_v.at[j])
            pltpu.sync_copy(items_hbm.at[pl.ds(src + j * CHUNK, CHUNK)],
                            idxi_v.at[j])
        ucopies = [pltpu.async_copy(utab_hbm.at[idxu_v.at[j]], rowsu_v.at[j],
                                    sem_u) for j in range(NCH)]
        icopies = [pltpu.async_copy(mtab_hbm.at[idxi_v.at[j]], rowsi_v.at[j],
                                    sem_i) for j in range(NCH)]
        outs = []
        for j in range(NCH):
            ucopies[j].wait()
            _convert_rows(rowsu_v, bfu_v, j)
            outs.append(pltpu.async_copy(
                bfu_v.at[j], uout_hbm.at[pl.ds(dst + j * CHUNK, CHUNK)],
                sem_out))
        for j in range(NCH):
            icopies[j].wait()
            _convert_rows(rowsi_v, bfi_v, j)
            outs.append(pltpu.async_copy(
                bfi_v.at[j], iout_hbm.at[pl.ds(dst + j * CHUNK, CHUNK)],
                sem_out))
        for c in outs:
            c.wait()
    return _gather_body


def _sc_gather(p, users, items, user_table, movie_table):
    mesh = plsc.VectorSubcoreMesh(core_axis_name="c", subcore_axis_name="s",
                                  num_cores=NC, num_subcores=NS)
    emb = jax.ShapeDtypeStruct((CB, EMB), jnp.bfloat16)
    run = pl.kernel(
        _make_gather_body(p),
        mesh=mesh,
        compiler_params=pltpu.CompilerParams(needs_layout_passes=False),
        out_type=[emb, emb],
        scratch_types=[
            pltpu.VMEM((NCH, CHUNK), jnp.int32),
            pltpu.VMEM((NCH, CHUNK), jnp.int32),
            pltpu.VMEM((NCH, CHUNK, EMB), jnp.float32),
            pltpu.VMEM((NCH, CHUNK, EMB), jnp.float32),
            pltpu.VMEM((NCH, CHUNK, EMB), jnp.bfloat16),
            pltpu.VMEM((NCH, CHUNK, EMB), jnp.bfloat16),
            pltpu.SemaphoreType.DMA,
            pltpu.SemaphoreType.DMA,
            pltpu.SemaphoreType.DMA,
        ],
    )
    return run(users, items, user_table, movie_table)


def _mlp_body(u_ref, i_ref, w1a_ref, w1b_ref, b1_ref, w2_ref, b2_ref,
              wout_ref, bout_ref, out_ref):
    u = u_ref[:].astype(jnp.float32)
    i = i_ref[:].astype(jnp.float32)
    h = jnp.dot(u, w1a_ref[:], preferred_element_type=jnp.float32)
    h = h + jnp.dot(i, w1b_ref[:], preferred_element_type=jnp.float32)
    h = jnp.maximum(h + b1_ref[:], 0.0)
    h = jnp.maximum(
        jnp.dot(h, w2_ref[:], preferred_element_type=jnp.float32) + b2_ref[:],
        0.0)
    out_ref[:] = (jnp.dot(h, wout_ref[:], preferred_element_type=jnp.float32)
                  + bout_ref[:])


def _tc_mlp(u_emb, i_emb, w1a, w1b, b1, W2, b2, Wout, bout, tile=2048):
    grid = (CB // tile,)
    row_spec = pl.BlockSpec((tile, EMB), lambda g: (g, 0))
    full = lambda shape: pl.BlockSpec(shape, lambda g: (0,) * len(shape))
    return pl.pallas_call(
        _mlp_body,
        grid=grid,
        in_specs=[
            row_spec, row_spec,
            full((EMB, 128)), full((EMB, 128)), full((1, 128)),
            full((128, 64)), full((1, 64)),
            full((64, 1)), full((1, 1)),
        ],
        out_specs=pl.BlockSpec((tile, 1), lambda g: (g, 0)),
        out_shape=jax.ShapeDtypeStruct((CB, 1), jnp.float32),
    )(u_emb, i_emb, w1a, w1b, b1.reshape(1, 128), W2, b2.reshape(1, 64),
      Wout, bout.reshape(1, 1))


@jax.jit
def kernel(users, items, user_table, movie_table, W1, b1, W2, b2, Wout, bout):
    perm = jnp.asarray(_PERM)
    w1a = W1[:EMB][perm]
    w1b = W1[EMB:][perm]
    embs = [_sc_gather(p, users, items, user_table, movie_table)
            for p in range(PIPE)]
    outs = [_tc_mlp(u, i, w1a, w1b, b1, W2, b2, Wout, bout)
            for u, i in embs]
    return jnp.concatenate(outs, axis=0)


# speedup vs baseline: 1.2468x; 1.2468x over previous
import functools

import jax
import jax.numpy as jnp
from jax import lax
from jax.experimental import pallas as pl
from jax.experimental.pallas import tpu as pltpu
from jax.experimental.pallas import tpu_sc as plsc

BATCH = 16384
EMB = 128
NC, NS = 2, 16
NW = NC * NS
CHUNK = 128
PIPE = 2
CB = BATCH // PIPE
B_PER_W = CB // NW
NCH = B_PER_W // CHUNK


def _make_gather_body(p):
    def _gather_body(users_hbm, items_hbm, utab_hbm, mtab_hbm,
                     uout_hbm, iout_hbm, idxu_v, idxi_v, rowsu_v, rowsi_v,
                     sem_u, sem_i, sem_out):
        wid = lax.axis_index("s") * NC + lax.axis_index("c")
        src = p * CB + wid * B_PER_W
        dst = wid * B_PER_W
        iu = pltpu.async_copy(users_hbm.at[pl.ds(src, B_PER_W)], idxu_v,
                              sem_u)
        ii = pltpu.async_copy(items_hbm.at[pl.ds(src, B_PER_W)], idxi_v,
                              sem_i)
        iu.wait()
        ii.wait()
        ucopies = [pltpu.async_copy(
            utab_hbm.at[idxu_v.at[pl.ds(j * CHUNK, CHUNK)]], rowsu_v.at[j],
            sem_u) for j in range(NCH)]
        icopies = [pltpu.async_copy(
            mtab_hbm.at[idxi_v.at[pl.ds(j * CHUNK, CHUNK)]], rowsi_v.at[j],
            sem_i) for j in range(NCH)]
        outs = []
        for j in range(NCH):
            ucopies[j].wait()
            outs.append(pltpu.async_copy(
                rowsu_v.at[j], uout_hbm.at[pl.ds(dst + j * CHUNK, CHUNK)],
                sem_out))
        for j in range(NCH):
            icopies[j].wait()
            outs.append(pltpu.async_copy(
                rowsi_v.at[j], iout_hbm.at[pl.ds(dst + j * CHUNK, CHUNK)],
                sem_out))
        for c in outs:
            c.wait()
    return _gather_body


def _sc_gather(p, users, items, user_table, movie_table):
    mesh = plsc.VectorSubcoreMesh(core_axis_name="c", subcore_axis_name="s",
                                  num_cores=NC, num_subcores=NS)
    emb = jax.ShapeDtypeStruct((CB, EMB), jnp.float32)
    run = pl.kernel(
        _make_gather_body(p),
        mesh=mesh,
        out_type=[emb, emb],
        scratch_types=[
            pltpu.VMEM((B_PER_W,), jnp.int32),
            pltpu.VMEM((B_PER_W,), jnp.int32),
            pltpu.VMEM((NCH, CHUNK, EMB), jnp.float32),
            pltpu.VMEM((NCH, CHUNK, EMB), jnp.float32),
            pltpu.SemaphoreType.DMA,
            pltpu.SemaphoreType.DMA,
            pltpu.SemaphoreType.DMA,
        ],
    )
    return run(users, items, user_table, movie_table)


def _mlp_body(*refs):
    if len(refs) == 11:
        _, u_ref, i_ref, w1a_ref, w1b_ref, b1_ref, w2_ref, b2_ref, \
            wout_ref, bout_ref, out_ref = refs
    else:
        u_ref, i_ref, w1a_ref, w1b_ref, b1_ref, w2_ref, b2_ref, \
            wout_ref, bout_ref, out_ref = refs
    h = jnp.dot(u_ref[:], w1a_ref[:], preferred_element_type=jnp.float32)
    h = h + jnp.dot(i_ref[:], w1b_ref[:], preferred_element_type=jnp.float32)
    h = jnp.maximum(h + b1_ref[:], 0.0)
    h = jnp.maximum(
        jnp.dot(h, w2_ref[:], preferred_element_type=jnp.float32) + b2_ref[:],
        0.0)
    out_ref[:] = (jnp.dot(h, wout_ref[:], preferred_element_type=jnp.float32)
                  + bout_ref[:])


def _tc_mlp(p, prev, u_emb, i_emb, W1, b1, W2, b2, Wout, bout, tile):
    grid = (CB // tile,)
    base = 0 if prev is None else p * (CB // tile)
    row_spec = pl.BlockSpec((tile, EMB), lambda g: (g, 0))
    full = lambda shape: pl.BlockSpec(shape, lambda g: (0,) * len(shape))
    in_specs = [
        row_spec, row_spec,
        pl.BlockSpec((EMB, 128), lambda g: (0, 0)),
        pl.BlockSpec((EMB, 128), lambda g: (1, 0)),
        full((1, 128)),
        full((128, 64)), full((1, 64)),
        full((64, 1)), full((1, 1)),
    ]
    args = (u_emb, i_emb, W1, W1, b1.reshape(1, 128), W2, b2.reshape(1, 64),
            Wout, bout.reshape(1, 1))
    aliases = {}
    if prev is not None:
        in_specs = [pl.BlockSpec(memory_space=pl.ANY)] + in_specs
        args = (prev,) + args
        aliases = {0: 0}
    return pl.pallas_call(
        _mlp_body,
        grid=grid,
        in_specs=in_specs,
        out_specs=pl.BlockSpec((tile, 1), lambda g: (base + g, 0)),
        out_shape=jax.ShapeDtypeStruct((CB, 1), jnp.float32),
        input_output_aliases=aliases,
    )(*args)


@jax.jit
def kernel(users, items, user_table, movie_table, W1, b1, W2, b2, Wout, bout):
    embs = [_sc_gather(p, users, items, user_table, movie_table)
            for p in range(PIPE)]
    outs = [_tc_mlp(p, None, u, i, W1, b1, W2, b2, Wout, bout, tile=2048)
            for p, (u, i) in enumerate(embs)]
    return jnp.concatenate(outs, axis=0)


# speedup vs baseline: 1.2935x; 1.0375x over previous
import functools

import jax
import jax.numpy as jnp
from jax import lax
from jax.experimental import pallas as pl
from jax.experimental.pallas import tpu as pltpu
from jax.experimental.pallas import tpu_sc as plsc

BATCH = 16384
EMB = 128
NC, NS = 2, 16
NW = NC * NS
CHUNK = 128
PIPE = 2
CB = BATCH // PIPE
B_PER_W = CB // NW
NCH = B_PER_W // CHUNK


def _make_gather_body(p):
    def _gather_body(users_hbm, items_hbm, utab_hbm, mtab_hbm,
                     uout_hbm, iout_hbm, idxu_v, idxi_v, rowsu_v, rowsi_v,
                     sem_u, sem_i, sem_out):
        wid = lax.axis_index("s") * NC + lax.axis_index("c")
        src = p * CB + wid * B_PER_W
        dst = wid * B_PER_W
        iu = pltpu.async_copy(users_hbm.at[pl.ds(src, B_PER_W)], idxu_v,
                              sem_u)
        ii = pltpu.async_copy(items_hbm.at[pl.ds(src, B_PER_W)], idxi_v,
                              sem_i)
        iu.wait()
        ii.wait()
        ucopies = [pltpu.async_copy(
            utab_hbm.at[idxu_v.at[pl.ds(j * CHUNK, CHUNK)]], rowsu_v.at[j],
            sem_u) for j in range(NCH)]
        icopies = [pltpu.async_copy(
            mtab_hbm.at[idxi_v.at[pl.ds(j * CHUNK, CHUNK)]], rowsi_v.at[j],
            sem_i) for j in range(NCH)]
        outs = []
        for j in range(NCH):
            ucopies[j].wait()
            outs.append(pltpu.async_copy(
                rowsu_v.at[j], uout_hbm.at[pl.ds(dst + j * CHUNK, CHUNK)],
                sem_out))
        for j in range(NCH):
            icopies[j].wait()
            outs.append(pltpu.async_copy(
                rowsi_v.at[j], iout_hbm.at[pl.ds(dst + j * CHUNK, CHUNK)],
                sem_out))
        for c in outs:
            c.wait()
    return _gather_body


def _sc_gather(p, users, items, user_table, movie_table):
    mesh = plsc.VectorSubcoreMesh(core_axis_name="c", subcore_axis_name="s",
                                  num_cores=NC, num_subcores=NS)
    emb = jax.ShapeDtypeStruct((CB, EMB), jnp.float32)
    run = pl.kernel(
        _make_gather_body(p),
        mesh=mesh,
        out_type=[emb, emb],
        scratch_types=[
            pltpu.VMEM((B_PER_W,), jnp.int32),
            pltpu.VMEM((B_PER_W,), jnp.int32),
            pltpu.VMEM((NCH, CHUNK, EMB), jnp.float32),
            pltpu.VMEM((NCH, CHUNK, EMB), jnp.float32),
            pltpu.SemaphoreType.DMA,
            pltpu.SemaphoreType.DMA,
            pltpu.SemaphoreType.DMA,
        ],
    )
    return run(users, items, user_table, movie_table)


def _mlp_body(*refs):
    if len(refs) == 11:
        _, u_ref, i_ref, w1a_ref, w1b_ref, b1_ref, w2_ref, b2_ref, \
            wout_ref, bout_ref, out_ref = refs
    else:
        u_ref, i_ref, w1a_ref, w1b_ref, b1_ref, w2_ref, b2_ref, \
            wout_ref, bout_ref, out_ref = refs
    h = jnp.dot(u_ref[:], w1a_ref[:], preferred_element_type=jnp.float32)
    h = h + jnp.dot(i_ref[:], w1b_ref[:], preferred_element_type=jnp.float32)
    h = jnp.maximum(h + b1_ref[:], 0.0)
    h = jnp.maximum(
        jnp.dot(h, w2_ref[:], preferred_element_type=jnp.float32) + b2_ref[:],
        0.0)
    out_ref[:] = (jnp.dot(h, wout_ref[:], preferred_element_type=jnp.float32)
                  + bout_ref[:])


def _tc_mlp(p, prev, u_emb, i_emb, W1, b1, W2, b2, Wout, bout, tile):
    grid = (CB // tile,)
    base = 0 if prev is None else p * (CB // tile)
    row_spec = pl.BlockSpec((tile, EMB), lambda g: (g, 0))
    full = lambda shape: pl.BlockSpec(shape, lambda g: (0,) * len(shape))
    in_specs = [
        row_spec, row_spec,
        pl.BlockSpec((EMB, 128), lambda g: (0, 0)),
        pl.BlockSpec((EMB, 128), lambda g: (1, 0)),
        full((1, 128)),
        full((128, 64)), full((1, 64)),
        full((64, 1)), full((1, 1)),
    ]
    args = (u_emb, i_emb, W1, W1, b1.reshape(1, 128), W2, b2.reshape(1, 64),
            Wout, bout.reshape(1, 1))
    aliases = {}
    if prev is not None:
        in_specs = [pl.BlockSpec(memory_space=pl.ANY)] + in_specs
        args = (prev,) + args
        aliases = {0: 0}
    return pl.pallas_call(
        _mlp_body,
        grid=grid,
        in_specs=in_specs,
        out_specs=pl.BlockSpec((tile, 1), lambda g: (base + g, 0)),
        out_shape=jax.ShapeDtypeStruct((CB, 1), jnp.float32),
        input_output_aliases=aliases,
    )(*args)


@jax.jit
def kernel(users, items, user_table, movie_table, W1, b1, W2, b2, Wout, bout):
    embs = [_sc_gather(p, users, items, user_table, movie_table)
            for p in range(PIPE)]
    outs = [_tc_mlp(p, None, u, i, W1, b1, W2, b2, Wout, bout, tile=4096)
            for p, (u, i) in enumerate(embs)]
    return jnp.concatenate(outs, axis=0)
